# baseline (speedup 1.0000x reference)
import jax
import jax.numpy as jnp
from jax.experimental import pallas as pl
from jax.experimental.pallas import tpu as pltpu

_ROWS = 2 * 8192
_COLS = 4096
_BLOCK_ROWS = 1016


def _copy_body(i_ref, o_ref):
    o_ref[...] = i_ref[...]


def kernel(x, bit, alpha):
    del bit, alpha
    x2 = x.reshape(_ROWS, _COLS)
    out = pl.pallas_call(
        _copy_body,
        grid=(-(-_ROWS // _BLOCK_ROWS),),
        in_specs=[pl.BlockSpec((_BLOCK_ROWS, _COLS), lambda i: (i, 0))],
        out_specs=pl.BlockSpec((_BLOCK_ROWS, _COLS), lambda i: (i, 0)),
        out_shape=jax.ShapeDtypeStruct((_ROWS, _COLS), x.dtype),
        compiler_params=pltpu.CompilerParams(skip_device_barrier=True, vmem_limit_bytes=100 * 1024 * 1024),
    )(x2)
    return out.reshape(x.shape)


# speedup vs baseline: 1.0009x; 1.0009x over previous
import jax
import jax.numpy as jnp
from jax.experimental import pallas as pl
from jax.experimental.pallas import tpu as pltpu

_ROWS = 2 * 8192
_COLS = 4096
_BLOCK_ROWS = 1008


def _copy_body(i_ref, o_ref):
    o_ref[...] = i_ref[...]


def kernel(x, bit, alpha):
    del bit, alpha
    x2 = x.reshape(_ROWS, _COLS)
    out = pl.pallas_call(
        _copy_body,
        grid=(-(-_ROWS // _BLOCK_ROWS),),
        in_specs=[pl.BlockSpec((_BLOCK_ROWS, _COLS), lambda i: (i, 0))],
        out_specs=pl.BlockSpec((_BLOCK_ROWS, _COLS), lambda i: (i, 0)),
        out_shape=jax.ShapeDtypeStruct((_ROWS, _COLS), x.dtype),
        compiler_params=pltpu.CompilerParams(vmem_limit_bytes=100 * 1024 * 1024),
    )(x2)
    return out.reshape(x.shape)


# speedup vs baseline: 1.0020x; 1.0010x over previous
import jax
import jax.numpy as jnp
from jax.experimental import pallas as pl
from jax.experimental.pallas import tpu as pltpu

_ROWS = 2 * 8192
_COLS = 4096
_CHUNK_ROWS = 1024
_NCH = _ROWS // _CHUNK_ROWS
_NBUF = 3


def _ring_body(i_ref, o_ref, bufs, lsem, ssem):
    def ld(i, slot):
        return pltpu.make_async_copy(
            i_ref.at[pl.ds(i * _CHUNK_ROWS, _CHUNK_ROWS)], bufs.at[slot], lsem.at[slot]
        )

    def st(i, slot):
        return pltpu.make_async_copy(
            bufs.at[slot], o_ref.at[pl.ds(i * _CHUNK_ROWS, _CHUNK_ROWS)], ssem.at[slot]
        )

    for b in range(_NBUF):
        ld(b, b).start()

    for i in range(_NCH):
        slot = i % _NBUF
        ld(i, slot).wait()
        st(i, slot).start()
        nxt = i + _NBUF
        if nxt < _NCH:
            st(i, slot).wait()
            ld(nxt, slot).start()

    for i in range(_NCH - _NBUF, _NCH):
        st(i, i % _NBUF).wait()


def kernel(x, bit, alpha):
    del bit, alpha
    x2 = x.reshape(_ROWS, _COLS)
    out = pl.pallas_call(
        _ring_body,
        in_specs=[pl.BlockSpec(memory_space=pl.ANY)],
        out_specs=pl.BlockSpec(memory_space=pl.ANY),
        out_shape=jax.ShapeDtypeStruct((_ROWS, _COLS), x.dtype),
        scratch_shapes=[
            pltpu.VMEM((_NBUF, _CHUNK_ROWS, _COLS), jnp.float32),
            pltpu.SemaphoreType.DMA((_NBUF,)),
            pltpu.SemaphoreType.DMA((_NBUF,)),
        ],
        compiler_params=pltpu.CompilerParams(vmem_limit_bytes=100 * 1024 * 1024),
    )(x2)
    return out.reshape(x.shape)
